# Initial kernel scaffold; baseline (speedup 1.0000x reference)
import jax
import jax.numpy as jnp
from jax.experimental import pallas as pl


def kernel(x, adj, w_embed, w_classify):
    raise NotImplementedError("write your pallas kernel here")


# speedup vs baseline: 2.4521x; 2.4521x over previous
import functools

import jax
import jax.numpy as jnp
from jax import lax
from jax.experimental import pallas as pl
from jax.experimental.pallas import tpu as pltpu
from jax.experimental.pallas import tpu_sc as plsc

_CB = 64
_NC = 2
_NS = 16
_LANES = 16
_CHUNK = 80


def _spmm_call(table_flat, rows, cols, nblocks):
    bn, cb = table_flat.shape
    n = bn // nblocks
    e = rows.shape[0]
    bpc = nblocks // _NC
    epw = e // _NS
    nchunks = epw // _CHUNK
    ra = (n // _NS) // 8 * 8
    rb = n - (_NS - 1) * ra

    mesh = plsc.VectorSubcoreMesh(core_axis_name="c", subcore_axis_name="s",
                                  num_cores=_NC, num_subcores=_NS)

    @functools.partial(
        pl.kernel,
        out_type=jax.ShapeDtypeStruct((nblocks, n, cb), jnp.float32),
        mesh=mesh,
        scratch_types=[
            pltpu.VMEM((_CHUNK,), jnp.int32),
            pltpu.VMEM((_CHUNK,), jnp.int32),
            pltpu.VMEM((_CHUNK, cb), jnp.float32),
            pltpu.VMEM((rb, cb), jnp.float32),
            pltpu.VMEM((rb, cb), jnp.float32),
            pltpu.VMEM_SHARED((n, cb), jnp.float32),
            pltpu.SemaphoreType.DMA,
        ],
        compiler_params=pltpu.CompilerParams(use_tc_tiling_on_sc=False),
    )
    def spmm_kernel(table, rows_r, cols_r, out, cidx, ridx, gbuf, zbuf, wbuf,
                    acc, gsem):
        cid = lax.axis_index("c")
        sid = lax.axis_index("s")
        ebase = sid * epw
        rbase = sid * ra
        last = sid == _NS - 1

        def zero_body(i, carry):
            r = i // (cb // _LANES)
            q = i % (cb // _LANES)
            zbuf[r, pl.ds(q * _LANES, _LANES)] = jnp.zeros((_LANES,),
                                                           jnp.float32)
            return carry

        lax.fori_loop(0, rb * (cb // _LANES), zero_body, 0)

        for bi in range(bpc):
            b = cid * bpc + bi

            @pl.when(jnp.logical_not(last))
            def _():
                pltpu.sync_copy(zbuf.at[pl.ds(0, ra)],
                                acc.at[pl.ds(rbase, ra)])

            @pl.when(last)
            def _():
                pltpu.sync_copy(zbuf, acc.at[pl.ds((_NS - 1) * ra, rb)])

            plsc.subcore_barrier()

            off = b * n

            def chunk_body(g, carry):
                base = ebase + g * _CHUNK
                pltpu.sync_copy(cols_r.at[pl.ds(base, _CHUNK)], cidx)
                pltpu.sync_copy(rows_r.at[pl.ds(base, _CHUNK)], ridx)
                for j in range(_CHUNK // _LANES):
                    sl = pl.ds(j * _LANES, _LANES)
                    cidx[sl] = cidx[sl] + off
                pltpu.async_copy(table.at[cidx], gbuf, gsem).wait()
                pltpu.sync_copy(gbuf, acc.at[ridx], add=True)
                return carry

            lax.fori_loop(0, nchunks, chunk_body, 0)
            plsc.subcore_barrier()

            @pl.when(jnp.logical_not(last))
            def _():
                pltpu.sync_copy(acc.at[pl.ds(rbase, ra)],
                                wbuf.at[pl.ds(0, ra)])
                pltpu.sync_copy(wbuf.at[pl.ds(0, ra)],
                                out.at[b, pl.ds(rbase, ra)])

            @pl.when(last)
            def _():
                pltpu.sync_copy(acc.at[pl.ds((_NS - 1) * ra, rb)], wbuf)
                pltpu.sync_copy(wbuf, out.at[b, pl.ds((_NS - 1) * ra, rb)])

    return spmm_kernel(table_flat, rows, cols)


def _embed_call(x, w_embed):
    n, feat = x.shape
    hid = w_embed.shape[1]
    nb = hid // _CB
    r = 1000

    def body(x_ref, w_ref, o_ref):
        y = jnp.maximum(
            jnp.dot(x_ref[...], w_ref[...],
                    preferred_element_type=jnp.float32), 0.0)
        for j in range(nb):
            o_ref[j] = y[:, j * _CB:(j + 1) * _CB]

    return pl.pallas_call(
        body,
        grid=(n // r,),
        in_specs=[
            pl.BlockSpec((r, feat), lambda i: (i, 0)),
            pl.BlockSpec((feat, hid), lambda i: (0, 0)),
        ],
        out_specs=pl.BlockSpec((nb, r, _CB), lambda i: (0, i, 0)),
        out_shape=jax.ShapeDtypeStruct((nb, n, _CB), jnp.float32),
    )(x, w_embed)


def _rs1_call(r0, b1, b2):
    nb0, n, cb = r0.shape
    r = 1000

    def body(r0_ref, b1_ref, b2_ref, o_ref):
        j = pl.program_id(1)
        v1 = b1_ref[0] - r0_ref[0]
        v2 = b2_ref[0] - b1_ref[0] - r0_ref[0]
        o_ref[0] = jnp.maximum(jnp.where(j < nb0, v1, v2), 0.0)

    im = lambda i, j: (j % nb0, i, 0)
    return pl.pallas_call(
        body,
        grid=(n // r, 2 * nb0),
        in_specs=[
            pl.BlockSpec((1, r, cb), im),
            pl.BlockSpec((1, r, cb), im),
            pl.BlockSpec((1, r, cb), im),
        ],
        out_specs=pl.BlockSpec((1, r, cb), lambda i, j: (j, i, 0)),
        out_shape=jax.ShapeDtypeStruct((2 * nb0, n, cb), jnp.float32),
    )(r0, b1, b2)


def _final_call(r0, rs1, c1, c2, w_classify):
    n = r0.shape[1]
    cls = w_classify.shape[1]
    r = 1000

    def body(r0_ref, rs1_ref, c1_ref, c2_ref, w_ref, o_ref):
        blocks = [r0_ref[j] for j in range(2)]
        blocks += [rs1_ref[j] for j in range(4)]
        blocks += [jnp.maximum(c1_ref[j] - rs1_ref[j], 0.0) for j in range(4)]
        blocks += [jnp.maximum(c2_ref[j] - c1_ref[j] - rs1_ref[j], 0.0)
                   for j in range(4)]
        acc = jnp.zeros((r, cls), jnp.float32)
        for k, blk in enumerate(blocks):
            acc = acc + jnp.dot(blk, w_ref[k * _CB:(k + 1) * _CB, :],
                                preferred_element_type=jnp.float32)
        m = jnp.max(acc, axis=1, keepdims=True)
        ex = jnp.exp(acc - m)
        o_ref[...] = ex / jnp.sum(ex, axis=1, keepdims=True)

    return pl.pallas_call(
        body,
        grid=(n // r,),
        in_specs=[
            pl.BlockSpec((2, r, _CB), lambda i: (0, i, 0)),
            pl.BlockSpec((4, r, _CB), lambda i: (0, i, 0)),
            pl.BlockSpec((4, r, _CB), lambda i: (0, i, 0)),
            pl.BlockSpec((4, r, _CB), lambda i: (0, i, 0)),
            pl.BlockSpec(w_classify.shape, lambda i: (0, 0)),
        ],
        out_specs=pl.BlockSpec((r, cls), lambda i: (i, 0)),
        out_shape=jax.ShapeDtypeStruct((n, cls), jnp.float32),
    )(r0, rs1, c1, c2, w_classify)


def kernel(x, adj, w_embed, w_classify):
    rows = adj[0].astype(jnp.int32)
    cols = adj[1].astype(jnp.int32)
    r0 = _embed_call(x, w_embed)
    b1 = _spmm_call(r0.reshape(-1, _CB), rows, cols, 2)
    b2 = _spmm_call(b1.reshape(-1, _CB), rows, cols, 2)
    rs1 = _rs1_call(r0, b1, b2)
    c1 = _spmm_call(rs1.reshape(-1, _CB), rows, cols, 4)
    c2 = _spmm_call(c1.reshape(-1, _CB), rows, cols, 4)
    return _final_call(r0, rs1, c1, c2, w_classify)


# speedup vs baseline: 6.8833x; 2.8071x over previous
import functools

import jax
import jax.numpy as jnp
from jax import lax
from jax.experimental import pallas as pl
from jax.experimental.pallas import tpu as pltpu
from jax.experimental.pallas import tpu_sc as plsc

_CB = 64
_NC = 2
_NS = 16
_LANES = 16
_CHUNK = 80


_K = 5
_ZROWS = 320


def _spmm_call(table_flat, rows2d, cols2d, nblocks):
    bn, cb = table_flat.shape
    n = bn // nblocks
    e = rows2d.shape[0] * rows2d.shape[1]
    bpc = nblocks // _NC
    epw = e // _NS
    nchunks = epw // _CHUNK
    ngroups = nchunks // _K
    ra = (n // _NS) // 8 * 8
    rb = n - (_NS - 1) * ra

    mesh = plsc.VectorSubcoreMesh(core_axis_name="c", subcore_axis_name="s",
                                  num_cores=_NC, num_subcores=_NS)

    @functools.partial(
        pl.kernel,
        out_type=jax.ShapeDtypeStruct((nblocks, n, cb), jnp.float32),
        mesh=mesh,
        scratch_types=[
            pltpu.VMEM((nchunks, _CHUNK), jnp.int32),
            pltpu.VMEM((nchunks, _CHUNK), jnp.int32),
            pltpu.VMEM((_K, _CHUNK, cb), jnp.float32),
            pltpu.VMEM((_ZROWS, cb), jnp.float32),
            pltpu.VMEM_SHARED((n, cb), jnp.float32),
            pltpu.SemaphoreType.DMA((_K,)),
        ],
        compiler_params=pltpu.CompilerParams(use_tc_tiling_on_sc=False),
    )
    def spmm_kernel(table, rows_r, cols_r, out, cidx, ridx, gbuf,
                    zbuf, acc, gsem):
        cid = lax.axis_index("c")
        sid = lax.axis_index("s")
        rbase = sid * ra
        last = sid == _NS - 1

        if True:
            def zfill(i, carry):
                r = i // (cb // _LANES)
                q = i % (cb // _LANES)
                zbuf[r, pl.ds(q * _LANES, _LANES)] = jnp.zeros(
                    (_LANES,), jnp.float32)
                return carry

            lax.fori_loop(0, _ZROWS * (cb // _LANES), zfill, 0)

            pltpu.sync_copy(cols_r.at[pl.ds(sid * nchunks, nchunks)], cidx)
            pltpu.sync_copy(rows_r.at[pl.ds(sid * nchunks, nchunks)], ridx)

            prev_off = 0
            for bi in range(bpc):
                b = cid * bpc + bi
                off = b * n
                delta = off - prev_off
                prev_off = off

                def adj_body(i, carry):
                    r = i // (_CHUNK // _LANES)
                    q = i % (_CHUNK // _LANES)
                    sl = pl.ds(q * _LANES, _LANES)
                    cidx[r, sl] = cidx[r, sl] + delta
                    return carry

                lax.fori_loop(0, nchunks * (_CHUNK // _LANES), adj_body, 0)

                @pl.when(jnp.logical_not(last))
                def _():
                    pltpu.sync_copy(zbuf, acc.at[pl.ds(rbase, _ZROWS)])
                    pltpu.sync_copy(zbuf.at[pl.ds(0, ra - _ZROWS)],
                                    acc.at[pl.ds(rbase + _ZROWS,
                                                 ra - _ZROWS)])

                @pl.when(last)
                def _():
                    lb = (_NS - 1) * ra
                    pltpu.sync_copy(zbuf, acc.at[pl.ds(lb, _ZROWS)])
                    pltpu.sync_copy(zbuf.at[pl.ds(0, rb - _ZROWS)],
                                    acc.at[pl.ds(lb + _ZROWS,
                                                 rb - _ZROWS)])

                plsc.subcore_barrier()

                def group_body(g, carry):
                    descs = []
                    for k in range(_K):
                        c = g * _K + k
                        descs.append(pltpu.async_copy(
                            table.at[cidx.at[c]], gbuf.at[k], gsem.at[k]))
                    for k in range(_K):
                        c = g * _K + k
                        descs[k].wait()
                        pltpu.sync_copy(gbuf.at[k], acc.at[ridx.at[c]],
                                        add=True)
                    return carry

                lax.fori_loop(0, ngroups, group_body, 0)
                plsc.subcore_barrier()

                @pl.when(jnp.logical_not(last))
                def _():
                    pltpu.sync_copy(acc.at[pl.ds(rbase, ra)],
                                    out.at[b, pl.ds(rbase, ra)])

                @pl.when(last)
                def _():
                    pltpu.sync_copy(acc.at[pl.ds((_NS - 1) * ra, rb)],
                                    out.at[b, pl.ds((_NS - 1) * ra, rb)])

    return spmm_kernel(table_flat, rows2d, cols2d)


def _embed_call(x, w_embed):
    n, feat = x.shape
    hid = w_embed.shape[1]
    nb = hid // _CB
    r = 1000

    def body(x_ref, w_ref, o_ref):
        y = jnp.maximum(
            jnp.dot(x_ref[...], w_ref[...],
                    preferred_element_type=jnp.float32), 0.0)
        for j in range(nb):
            o_ref[j] = y[:, j * _CB:(j + 1) * _CB]

    return pl.pallas_call(
        body,
        grid=(n // r,),
        in_specs=[
            pl.BlockSpec((r, feat), lambda i: (i, 0)),
            pl.BlockSpec((feat, hid), lambda i: (0, 0)),
        ],
        out_specs=pl.BlockSpec((nb, r, _CB), lambda i: (0, i, 0)),
        out_shape=jax.ShapeDtypeStruct((nb, n, _CB), jnp.float32),
    )(x, w_embed)


def _rs1_call(r0, b1, b2):
    nb0, n, cb = r0.shape
    r = 1000

    def body(r0_ref, b1_ref, b2_ref, o_ref):
        j = pl.program_id(1)
        v1 = b1_ref[0] - r0_ref[0]
        v2 = b2_ref[0] - b1_ref[0] - r0_ref[0]
        o_ref[0] = jnp.maximum(jnp.where(j < nb0, v1, v2), 0.0)

    im = lambda i, j: (j % nb0, i, 0)
    return pl.pallas_call(
        body,
        grid=(n // r, 2 * nb0),
        in_specs=[
            pl.BlockSpec((1, r, cb), im),
            pl.BlockSpec((1, r, cb), im),
            pl.BlockSpec((1, r, cb), im),
        ],
        out_specs=pl.BlockSpec((1, r, cb), lambda i, j: (j, i, 0)),
        out_shape=jax.ShapeDtypeStruct((2 * nb0, n, cb), jnp.float32),
    )(r0, b1, b2)


def _final_call(r0, rs1, c1, c2, w_classify):
    n = r0.shape[1]
    cls = w_classify.shape[1]
    r = 1000

    def body(r0_ref, rs1_ref, c1_ref, c2_ref, w_ref, o_ref):
        blocks = [r0_ref[j] for j in range(2)]
        blocks += [rs1_ref[j] for j in range(4)]
        blocks += [jnp.maximum(c1_ref[j] - rs1_ref[j], 0.0) for j in range(4)]
        blocks += [jnp.maximum(c2_ref[j] - c1_ref[j] - rs1_ref[j], 0.0)
                   for j in range(4)]
        acc = jnp.zeros((r, cls), jnp.float32)
        for k, blk in enumerate(blocks):
            acc = acc + jnp.dot(blk, w_ref[k * _CB:(k + 1) * _CB, :],
                                preferred_element_type=jnp.float32)
        m = jnp.max(acc, axis=1, keepdims=True)
        ex = jnp.exp(acc - m)
        o_ref[...] = ex / jnp.sum(ex, axis=1, keepdims=True)

    return pl.pallas_call(
        body,
        grid=(n // r,),
        in_specs=[
            pl.BlockSpec((2, r, _CB), lambda i: (0, i, 0)),
            pl.BlockSpec((4, r, _CB), lambda i: (0, i, 0)),
            pl.BlockSpec((4, r, _CB), lambda i: (0, i, 0)),
            pl.BlockSpec((4, r, _CB), lambda i: (0, i, 0)),
            pl.BlockSpec(w_classify.shape, lambda i: (0, 0)),
        ],
        out_specs=pl.BlockSpec((r, cls), lambda i: (i, 0)),
        out_shape=jax.ShapeDtypeStruct((n, cls), jnp.float32),
    )(r0, rs1, c1, c2, w_classify)


def kernel(x, adj, w_embed, w_classify):
    n = x.shape[0]
    rows2d = adj[0].astype(jnp.int32).reshape(-1, _CHUNK)
    cols2d = adj[1].astype(jnp.int32).reshape(-1, _CHUNK)
    r0 = _embed_call(x, w_embed)
    b1 = _spmm_call(r0.reshape(-1, _CB), rows2d, cols2d, 2)
    b2 = _spmm_call(b1.reshape(-1, _CB), rows2d, cols2d, 2)
    rs1 = _rs1_call(r0, b1, b2)
    c1 = _spmm_call(rs1.reshape(-1, _CB), rows2d, cols2d, 4)
    c2 = _spmm_call(c1.reshape(-1, _CB), rows2d, cols2d, 4)
    return _final_call(r0, rs1, c1, c2, w_classify)


# speedup vs baseline: 9.4313x; 1.3702x over previous
import functools

import jax
import jax.numpy as jnp
from jax import lax
from jax.experimental import pallas as pl
from jax.experimental.pallas import tpu as pltpu
from jax.experimental.pallas import tpu_sc as plsc

_CB = 64
_NC = 2
_NS = 16
_LANES = 16
_CHUNK = 80


_K = 4
_NSLOTS = 2 * _K
_ZROWS = 208


def _spmm_call(table_flat, rows2d, cols2d, nblocks):
    bn, cb = table_flat.shape
    n = bn // nblocks
    e = rows2d.shape[0] * rows2d.shape[1]
    bpc = nblocks // _NC
    epw = e // _NS
    nchunks = epw // _CHUNK
    ngroups = nchunks // _K
    ra = (n // _NS) // 8 * 8
    rb = n - (_NS - 1) * ra
    niters = (nchunks - _K + _NSLOTS - 1) // _NSLOTS
    tail = nchunks - niters * _NSLOTS
    assert 0 <= tail <= _K

    mesh = plsc.VectorSubcoreMesh(core_axis_name="c", subcore_axis_name="s",
                                  num_cores=_NC, num_subcores=_NS)

    @functools.partial(
        pl.kernel,
        out_type=jax.ShapeDtypeStruct((nblocks, n, cb), jnp.float32),
        mesh=mesh,
        scratch_types=[
            pltpu.VMEM((nchunks, _CHUNK), jnp.int32),
            pltpu.VMEM((nchunks, _CHUNK), jnp.int32),
            pltpu.VMEM((_NSLOTS * _CHUNK, cb), jnp.float32),
            pltpu.VMEM_SHARED((n, cb), jnp.float32),
            pltpu.SemaphoreType.DMA((_NSLOTS,)),
            pltpu.SemaphoreType.DMA((_NSLOTS,)),
        ],
        compiler_params=pltpu.CompilerParams(use_tc_tiling_on_sc=False),
    )
    def spmm_kernel(table, rows_r, cols_r, out, cidx, ridx, gbuf, acc, gsem,
                    ssem):
        cid = lax.axis_index("c")
        sid = lax.axis_index("s")
        rbase = sid * ra
        last = sid == _NS - 1

        def slot(k):
            return gbuf.at[pl.ds(k * _CHUNK, _CHUNK)]

        def drain_gsem(k):
            pltpu.make_async_copy(
                table.at[pl.ds(0, _CHUNK)], slot(k), gsem.at[k]).wait()

        pltpu.sync_copy(cols_r.at[pl.ds(sid * nchunks, nchunks)], cidx)
        pltpu.sync_copy(rows_r.at[pl.ds(sid * nchunks, nchunks)], ridx)

        prev_off = 0
        for bi in range(bpc):
            b = cid * bpc + bi
            off = b * n
            delta = off - prev_off
            prev_off = off

            def adj_body(i, carry):
                r = i // (_CHUNK // _LANES)
                q = i % (_CHUNK // _LANES)
                sl = pl.ds(q * _LANES, _LANES)
                cidx[r, sl] = cidx[r, sl] + delta
                return carry

            lax.fori_loop(0, nchunks * (_CHUNK // _LANES), adj_body, 0)

            def zfill(i, carry):
                r = i // (cb // _LANES)
                q = i % (cb // _LANES)
                gbuf[r, pl.ds(q * _LANES, _LANES)] = jnp.zeros(
                    (_LANES,), jnp.float32)
                return carry

            lax.fori_loop(0, _ZROWS * (cb // _LANES), zfill, 0)
            zsrc = gbuf.at[pl.ds(0, _ZROWS)]

            @pl.when(jnp.logical_not(last))
            def _():
                for z in range(ra // _ZROWS):
                    pltpu.sync_copy(
                        zsrc, acc.at[pl.ds(rbase + z * _ZROWS, _ZROWS)])

            @pl.when(last)
            def _():
                lbase = (_NS - 1) * ra
                for z in range(rb // _ZROWS):
                    pltpu.sync_copy(
                        zsrc, acc.at[pl.ds(lbase + z * _ZROWS, _ZROWS)])
                rem = rb % _ZROWS
                if rem:
                    pltpu.sync_copy(
                        gbuf.at[pl.ds(0, rem)],
                        acc.at[pl.ds(lbase + (rb // _ZROWS) * _ZROWS, rem)])

            plsc.subcore_barrier()

            for k in range(_K):
                pltpu.async_copy(table.at[cidx.at[k]], slot(k), gsem.at[k])

            def pipe_body(pp, carry):
                sa, sb = [], []
                for k in range(_K):
                    c = pp * _NSLOTS + _K + k
                    pltpu.async_copy(
                        table.at[cidx.at[c]], slot(_K + k), gsem.at[_K + k])
                for k in range(_K):
                    c = pp * _NSLOTS + k
                    drain_gsem(k)
                    sa.append(pltpu.async_copy(
                        slot(k), acc.at[ridx.at[c]], ssem.at[k], add=True))
                for k in range(_K):
                    c = pp * _NSLOTS + _K + k
                    drain_gsem(_K + k)
                    sb.append(pltpu.async_copy(
                        slot(_K + k), acc.at[ridx.at[c]], ssem.at[_K + k],
                        add=True))
                for k in range(_K):
                    c = jnp.minimum((pp + 1) * _NSLOTS + k, nchunks - 1)
                    sa[k].wait()
                    pltpu.async_copy(table.at[cidx.at[c]], slot(k),
                                     gsem.at[k])
                for k in range(_K):
                    sb[k].wait()
                return carry

            lax.fori_loop(0, niters, pipe_body, 0)

            for k in range(_K):
                drain_gsem(k)
            for k in range(tail):
                pltpu.sync_copy(slot(k),
                                acc.at[ridx.at[niters * _NSLOTS + k]],
                                add=True)

            plsc.subcore_barrier()

            @pl.when(jnp.logical_not(last))
            def _():
                pltpu.sync_copy(acc.at[pl.ds(rbase, ra)],
                                out.at[b, pl.ds(rbase, ra)])

            @pl.when(last)
            def _():
                pltpu.sync_copy(acc.at[pl.ds((_NS - 1) * ra, rb)],
                                out.at[b, pl.ds((_NS - 1) * ra, rb)])

    return spmm_kernel(table_flat, rows2d, cols2d)


def _embed_call(x, w_embed):
    n, feat = x.shape
    hid = w_embed.shape[1]
    nb = hid // _CB
    r = 1000

    def body(x_ref, w_ref, o_ref):
        y = jnp.maximum(
            jnp.dot(x_ref[...], w_ref[...],
                    preferred_element_type=jnp.float32), 0.0)
        for j in range(nb):
            o_ref[j] = y[:, j * _CB:(j + 1) * _CB]

    return pl.pallas_call(
        body,
        grid=(n // r,),
        in_specs=[
            pl.BlockSpec((r, feat), lambda i: (i, 0)),
            pl.BlockSpec((feat, hid), lambda i: (0, 0)),
        ],
        out_specs=pl.BlockSpec((nb, r, _CB), lambda i: (0, i, 0)),
        out_shape=jax.ShapeDtypeStruct((nb, n, _CB), jnp.float32),
    )(x, w_embed)


def _rs1_call(r0, b1, b2):
    nb0, n, cb = r0.shape
    r = 1000

    def body(r0_ref, b1_ref, b2_ref, o_ref):
        j = pl.program_id(1)
        v1 = b1_ref[0] - r0_ref[0]
        v2 = b2_ref[0] - b1_ref[0] - r0_ref[0]
        o_ref[0] = jnp.maximum(jnp.where(j < nb0, v1, v2), 0.0)

    im = lambda i, j: (j % nb0, i, 0)
    return pl.pallas_call(
        body,
        grid=(n // r, 2 * nb0),
        in_specs=[
            pl.BlockSpec((1, r, cb), im),
            pl.BlockSpec((1, r, cb), im),
            pl.BlockSpec((1, r, cb), im),
        ],
        out_specs=pl.BlockSpec((1, r, cb), lambda i, j: (j, i, 0)),
        out_shape=jax.ShapeDtypeStruct((2 * nb0, n, cb), jnp.float32),
    )(r0, b1, b2)


def _final_call(r0, rs1, c1, c2, w_classify):
    n = r0.shape[1]
    cls = w_classify.shape[1]
    r = 1000

    def body(r0_ref, rs1_ref, c1_ref, c2_ref, w_ref, o_ref):
        blocks = [r0_ref[j] for j in range(2)]
        blocks += [rs1_ref[j] for j in range(4)]
        blocks += [jnp.maximum(c1_ref[j] - rs1_ref[j], 0.0) for j in range(4)]
        blocks += [jnp.maximum(c2_ref[j] - c1_ref[j] - rs1_ref[j], 0.0)
                   for j in range(4)]
        acc = jnp.zeros((r, cls), jnp.float32)
        for k, blk in enumerate(blocks):
            acc = acc + jnp.dot(blk, w_ref[k * _CB:(k + 1) * _CB, :],
                                preferred_element_type=jnp.float32)
        m = jnp.max(acc, axis=1, keepdims=True)
        ex = jnp.exp(acc - m)
        o_ref[...] = ex / jnp.sum(ex, axis=1, keepdims=True)

    return pl.pallas_call(
        body,
        grid=(n // r,),
        in_specs=[
            pl.BlockSpec((2, r, _CB), lambda i: (0, i, 0)),
            pl.BlockSpec((4, r, _CB), lambda i: (0, i, 0)),
            pl.BlockSpec((4, r, _CB), lambda i: (0, i, 0)),
            pl.BlockSpec((4, r, _CB), lambda i: (0, i, 0)),
            pl.BlockSpec(w_classify.shape, lambda i: (0, 0)),
        ],
        out_specs=pl.BlockSpec((r, cls), lambda i: (i, 0)),
        out_shape=jax.ShapeDtypeStruct((n, cls), jnp.float32),
    )(r0, rs1, c1, c2, w_classify)


def kernel(x, adj, w_embed, w_classify):
    n = x.shape[0]
    rows2d = adj[0].astype(jnp.int32).reshape(-1, _CHUNK)
    cols2d = adj[1].astype(jnp.int32).reshape(-1, _CHUNK)
    r0 = _embed_call(x, w_embed)
    b1 = _spmm_call(r0.reshape(-1, _CB), rows2d, cols2d, 2)
    b2 = _spmm_call(b1.reshape(-1, _CB), rows2d, cols2d, 2)
    rs1 = _rs1_call(r0, b1, b2)
    c1 = _spmm_call(rs1.reshape(-1, _CB), rows2d, cols2d, 4)
    c2 = _spmm_call(c1.reshape(-1, _CB), rows2d, cols2d, 4)
    return _final_call(r0, rs1, c1, c2, w_classify)


# speedup vs baseline: 9.5291x; 1.0104x over previous
import functools

import jax
import jax.numpy as jnp
from jax import lax
from jax.experimental import pallas as pl
from jax.experimental.pallas import tpu as pltpu
from jax.experimental.pallas import tpu_sc as plsc

_CB = 64
_NC = 2
_NS = 16
_LANES = 16
_CHUNK = 80


_K = 4
_NSLOTS = 2 * _K
_ZROWS = 208


def _spmm_call(table_flat, rows2d, cols2d, nblocks):
    bn, cb = table_flat.shape
    n = bn // nblocks
    e = rows2d.shape[0] * rows2d.shape[1]
    bpc = nblocks // _NC
    epw = e // _NS
    nchunks = epw // _CHUNK
    ngroups = nchunks // _K
    ra = (n // _NS) // 8 * 8
    rb = n - (_NS - 1) * ra
    niters = (nchunks - _K + _NSLOTS - 1) // _NSLOTS
    tail = nchunks - niters * _NSLOTS
    assert 0 <= tail <= _K

    mesh = plsc.VectorSubcoreMesh(core_axis_name="c", subcore_axis_name="s",
                                  num_cores=_NC, num_subcores=_NS)

    @functools.partial(
        pl.kernel,
        out_type=(jax.ShapeDtypeStruct((nblocks * n, cb), jnp.float32),
                  jax.ShapeDtypeStruct((nblocks * n, cb), jnp.float32)),
        mesh=mesh,
        scratch_types=[
            pltpu.VMEM((nchunks, _CHUNK), jnp.int32),
            pltpu.VMEM((nchunks, _CHUNK), jnp.int32),
            pltpu.VMEM((_NSLOTS * _CHUNK, cb), jnp.float32),
            pltpu.VMEM_SHARED((n, cb), jnp.float32),
            pltpu.SemaphoreType.DMA((_NSLOTS,)),
            pltpu.SemaphoreType.DMA((_NSLOTS,)),
        ],
        compiler_params=pltpu.CompilerParams(use_tc_tiling_on_sc=False),
    )
    def spmm_kernel(table, rows_r, cols_r, out1, out2, cidx, ridx, gbuf, acc,
                    gsem, ssem):
        cid = lax.axis_index("c")
        sid = lax.axis_index("s")
        rbase = sid * ra
        last = sid == _NS - 1

        def slot(k):
            return gbuf.at[pl.ds(k * _CHUNK, _CHUNK)]

        def drain_gsem(k):
            pltpu.make_async_copy(
                table.at[pl.ds(0, _CHUNK)], slot(k), gsem.at[k]).wait()

        pltpu.sync_copy(cols_r.at[pl.ds(sid * nchunks, nchunks)], cidx)
        pltpu.sync_copy(rows_r.at[pl.ds(sid * nchunks, nchunks)], ridx)

        prev_off = 0
        for src, dst in ((table, out1), (out1, out2)):
          for bi in range(bpc):
            b = cid * bpc + bi
            off = b * n
            delta = off - prev_off
            prev_off = off

            def adj_body(i, carry):
                r = i // (_CHUNK // _LANES)
                q = i % (_CHUNK // _LANES)
                sl = pl.ds(q * _LANES, _LANES)
                cidx[r, sl] = cidx[r, sl] + delta
                return carry

            @pl.when(delta != 0)
            def _():
                lax.fori_loop(0, nchunks * (_CHUNK // _LANES), adj_body, 0)

            def zfill(i, carry):
                r = i // (cb // _LANES)
                q = i % (cb // _LANES)
                gbuf[r, pl.ds(q * _LANES, _LANES)] = jnp.zeros(
                    (_LANES,), jnp.float32)
                return carry

            lax.fori_loop(0, _ZROWS * (cb // _LANES), zfill, 0)
            zsrc = gbuf.at[pl.ds(0, _ZROWS)]

            @pl.when(jnp.logical_not(last))
            def _():
                for z in range(ra // _ZROWS):
                    pltpu.sync_copy(
                        zsrc, acc.at[pl.ds(rbase + z * _ZROWS, _ZROWS)])

            @pl.when(last)
            def _():
                lbase = (_NS - 1) * ra
                for z in range(rb // _ZROWS):
                    pltpu.sync_copy(
                        zsrc, acc.at[pl.ds(lbase + z * _ZROWS, _ZROWS)])
                rem = rb % _ZROWS
                if rem:
                    pltpu.sync_copy(
                        gbuf.at[pl.ds(0, rem)],
                        acc.at[pl.ds(lbase + (rb // _ZROWS) * _ZROWS, rem)])

            plsc.subcore_barrier()

            for k in range(_K):
                pltpu.async_copy(src.at[cidx.at[k]], slot(k), gsem.at[k])

            def pipe_body(pp, carry):
                sa, sb = [], []
                for k in range(_K):
                    c = pp * _NSLOTS + _K + k
                    pltpu.async_copy(
                        src.at[cidx.at[c]], slot(_K + k), gsem.at[_K + k])
                for k in range(_K):
                    c = pp * _NSLOTS + k
                    drain_gsem(k)
                    sa.append(pltpu.async_copy(
                        slot(k), acc.at[ridx.at[c]], ssem.at[k], add=True))
                for k in range(_K):
                    c = pp * _NSLOTS + _K + k
                    drain_gsem(_K + k)
                    sb.append(pltpu.async_copy(
                        slot(_K + k), acc.at[ridx.at[c]], ssem.at[_K + k],
                        add=True))
                for k in range(_K):
                    c = jnp.minimum((pp + 1) * _NSLOTS + k, nchunks - 1)
                    sa[k].wait()
                    pltpu.async_copy(src.at[cidx.at[c]], slot(k),
                                     gsem.at[k])
                for k in range(_K):
                    sb[k].wait()
                return carry

            lax.fori_loop(0, niters, pipe_body, 0)

            for k in range(_K):
                drain_gsem(k)
            for k in range(tail):
                pltpu.sync_copy(slot(k),
                                acc.at[ridx.at[niters * _NSLOTS + k]],
                                add=True)

            plsc.subcore_barrier()

            @pl.when(jnp.logical_not(last))
            def _():
                pltpu.sync_copy(acc.at[pl.ds(rbase, ra)],
                                dst.at[pl.ds(off + rbase, ra)])

            @pl.when(last)
            def _():
                pltpu.sync_copy(acc.at[pl.ds((_NS - 1) * ra, rb)],
                                dst.at[pl.ds(off + (_NS - 1) * ra, rb)])

    o1, o2 = spmm_kernel(table_flat, rows2d, cols2d)
    return (o1.reshape(nblocks, n, cb), o2.reshape(nblocks, n, cb))


def _embed_call(x, w_embed):
    n, feat = x.shape
    hid = w_embed.shape[1]
    nb = hid // _CB
    r = 1000

    def body(x_ref, w_ref, o_ref):
        y = jnp.maximum(
            jnp.dot(x_ref[...], w_ref[...],
                    preferred_element_type=jnp.float32), 0.0)
        for j in range(nb):
            o_ref[j] = y[:, j * _CB:(j + 1) * _CB]

    return pl.pallas_call(
        body,
        grid=(n // r,),
        in_specs=[
            pl.BlockSpec((r, feat), lambda i: (i, 0)),
            pl.BlockSpec((feat, hid), lambda i: (0, 0)),
        ],
        out_specs=pl.BlockSpec((nb, r, _CB), lambda i: (0, i, 0)),
        out_shape=jax.ShapeDtypeStruct((nb, n, _CB), jnp.float32),
    )(x, w_embed)


def _rs1_call(r0, b1, b2):
    nb0, n, cb = r0.shape
    r = 1000

    def body(r0_ref, b1_ref, b2_ref, o_ref):
        j = pl.program_id(1)
        v1 = b1_ref[0] - r0_ref[0]
        v2 = b2_ref[0] - b1_ref[0] - r0_ref[0]
        o_ref[0] = jnp.maximum(jnp.where(j < nb0, v1, v2), 0.0)

    im = lambda i, j: (j % nb0, i, 0)
    return pl.pallas_call(
        body,
        grid=(n // r, 2 * nb0),
        in_specs=[
            pl.BlockSpec((1, r, cb), im),
            pl.BlockSpec((1, r, cb), im),
            pl.BlockSpec((1, r, cb), im),
        ],
        out_specs=pl.BlockSpec((1, r, cb), lambda i, j: (j, i, 0)),
        out_shape=jax.ShapeDtypeStruct((2 * nb0, n, cb), jnp.float32),
    )(r0, b1, b2)


def _final_call(r0, rs1, c1, c2, w_classify):
    n = r0.shape[1]
    cls = w_classify.shape[1]
    r = 1000

    def body(r0_ref, rs1_ref, c1_ref, c2_ref, w_ref, o_ref):
        blocks = [r0_ref[j] for j in range(2)]
        blocks += [rs1_ref[j] for j in range(4)]
        blocks += [jnp.maximum(c1_ref[j] - rs1_ref[j], 0.0) for j in range(4)]
        blocks += [jnp.maximum(c2_ref[j] - c1_ref[j] - rs1_ref[j], 0.0)
                   for j in range(4)]
        acc = jnp.zeros((r, cls), jnp.float32)
        for k, blk in enumerate(blocks):
            acc = acc + jnp.dot(blk, w_ref[k * _CB:(k + 1) * _CB, :],
                                preferred_element_type=jnp.float32)
        m = jnp.max(acc, axis=1, keepdims=True)
        ex = jnp.exp(acc - m)
        o_ref[...] = ex / jnp.sum(ex, axis=1, keepdims=True)

    return pl.pallas_call(
        body,
        grid=(n // r,),
        in_specs=[
            pl.BlockSpec((2, r, _CB), lambda i: (0, i, 0)),
            pl.BlockSpec((4, r, _CB), lambda i: (0, i, 0)),
            pl.BlockSpec((4, r, _CB), lambda i: (0, i, 0)),
            pl.BlockSpec((4, r, _CB), lambda i: (0, i, 0)),
            pl.BlockSpec(w_classify.shape, lambda i: (0, 0)),
        ],
        out_specs=pl.BlockSpec((r, cls), lambda i: (i, 0)),
        out_shape=jax.ShapeDtypeStruct((n, cls), jnp.float32),
    )(r0, rs1, c1, c2, w_classify)


def kernel(x, adj, w_embed, w_classify):
    n = x.shape[0]
    rows2d = adj[0].astype(jnp.int32).reshape(-1, _CHUNK)
    cols2d = adj[1].astype(jnp.int32).reshape(-1, _CHUNK)
    r0 = _embed_call(x, w_embed)
    b1, b2 = _spmm_call(r0.reshape(-1, _CB), rows2d, cols2d, 2)
    rs1 = _rs1_call(r0, b1, b2)
    c1, c2 = _spmm_call(rs1.reshape(-1, _CB), rows2d, cols2d, 4)
    return _final_call(r0, rs1, c1, c2, w_classify)


# speedup vs baseline: 9.7529x; 1.0235x over previous
import functools

import jax
import jax.numpy as jnp
from jax import lax
from jax.experimental import pallas as pl
from jax.experimental.pallas import tpu as pltpu
from jax.experimental.pallas import tpu_sc as plsc

_CB = 64
_NC = 2
_NS = 16
_LANES = 16
_CHUNK = 80


_K = 4
_NSLOTS = 2 * _K
_ZROWS = 208


def _spmm_call(table3, rows2d, cols2d, nblocks):
    nblocks_, n, cb = table3.shape
    e = rows2d.shape[0] * rows2d.shape[1]
    bpc = nblocks // _NC
    epw = e // _NS
    nchunks = epw // _CHUNK
    ra = (n // _NS) // 8 * 8
    rb = n - (_NS - 1) * ra
    niters = (nchunks - _K + _NSLOTS - 1) // _NSLOTS
    tail = nchunks - niters * _NSLOTS
    assert 0 <= tail <= _K

    mesh = plsc.VectorSubcoreMesh(core_axis_name="c", subcore_axis_name="s",
                                  num_cores=_NC, num_subcores=_NS)

    @functools.partial(
        pl.kernel,
        out_type=(jax.ShapeDtypeStruct((nblocks, n, cb), jnp.float32),
                  jax.ShapeDtypeStruct((nblocks, n, cb), jnp.float32)),
        mesh=mesh,
        scratch_types=[
            pltpu.VMEM((nchunks, _CHUNK), jnp.int32),
            pltpu.VMEM((nchunks, _CHUNK), jnp.int32),
            pltpu.VMEM((_NSLOTS * _CHUNK, cb), jnp.float32),
            pltpu.VMEM_SHARED((n, cb), jnp.float32),
            pltpu.SemaphoreType.DMA((_NSLOTS,)),
            pltpu.SemaphoreType.DMA((_NSLOTS,)),
        ],
        compiler_params=pltpu.CompilerParams(use_tc_tiling_on_sc=False),
    )
    def spmm_kernel(table, rows_r, cols_r, out1, out2, cidx, ridx, gbuf, acc,
                    gsem, ssem):
        cid = lax.axis_index("c")
        sid = lax.axis_index("s")
        rbase = sid * ra
        last = sid == _NS - 1

        def slot(k):
            return gbuf.at[pl.ds(k * _CHUNK, _CHUNK)]

        def drain_gsem(k):
            pltpu.make_async_copy(
                table.at[0].at[pl.ds(0, _CHUNK)], slot(k), gsem.at[k]).wait()

        def drain_ssem(k):
            pltpu.make_async_copy(
                table.at[0].at[pl.ds(0, _CHUNK)], slot(k), ssem.at[k]).wait()

        pltpu.sync_copy(cols_r.at[pl.ds(sid * nchunks, nchunks)], cidx)
        pltpu.sync_copy(rows_r.at[pl.ds(sid * nchunks, nchunks)], ridx)

        for phase, (src, dst) in enumerate(((table, out1), (out1, out2))):
          for bi in range(bpc):
            b = cid * bpc + bi

            def gfire(c, k):
                pltpu.async_copy(src.at[b].at[cidx.at[c]], slot(k),
                                 gsem.at[k])

            if phase == 1 and bi == 0:
                plsc.subcore_barrier()

            for k in range(_K):
                gfire(k, _K + k)

            def zfill(i, carry):
                r = i // (cb // _LANES)
                q = i % (cb // _LANES)
                gbuf[r, pl.ds(q * _LANES, _LANES)] = jnp.zeros(
                    (_LANES,), jnp.float32)
                return carry

            lax.fori_loop(0, _ZROWS * (cb // _LANES), zfill, 0)
            zsrc = gbuf.at[pl.ds(0, _ZROWS)]

            @pl.when(jnp.logical_not(last))
            def _():
                for z in range(ra // _ZROWS):
                    pltpu.sync_copy(
                        zsrc, acc.at[pl.ds(rbase + z * _ZROWS, _ZROWS)])

            @pl.when(last)
            def _():
                lbase = (_NS - 1) * ra
                for z in range(rb // _ZROWS):
                    pltpu.sync_copy(
                        zsrc, acc.at[pl.ds(lbase + z * _ZROWS, _ZROWS)])
                rem = rb % _ZROWS
                if rem:
                    pltpu.sync_copy(
                        gbuf.at[pl.ds(0, rem)],
                        acc.at[pl.ds(lbase + (rb // _ZROWS) * _ZROWS, rem)])

            plsc.subcore_barrier()

            def pipe_body(pp, carry):
                sa = []
                for k in range(_K):
                    @pl.when(pp > 0)
                    def _():
                        drain_ssem(k)
                    gfire(pp * _NSLOTS + _K + k, k)
                for k in range(_K):
                    c = pp * _NSLOTS + k
                    drain_gsem(_K + k)
                    sa.append(pltpu.async_copy(
                        slot(_K + k), acc.at[ridx.at[c]], ssem.at[_K + k],
                        add=True))
                for k in range(_K):
                    c = pp * _NSLOTS + _K + k
                    drain_gsem(k)
                    pltpu.async_copy(
                        slot(k), acc.at[ridx.at[c]], ssem.at[k], add=True)
                for k in range(_K):
                    c = jnp.minimum((pp + 1) * _NSLOTS + k, nchunks - 1)
                    sa[k].wait()
                    gfire(c, _K + k)
                return carry

            lax.fori_loop(0, niters, pipe_body, 0)

            for k in range(_K):
                drain_ssem(k)
                drain_gsem(_K + k)
            for k in range(tail):
                pltpu.sync_copy(slot(_K + k),
                                acc.at[ridx.at[niters * _NSLOTS + k]],
                                add=True)

            plsc.subcore_barrier()

            @pl.when(jnp.logical_not(last))
            def _():
                pltpu.sync_copy(acc.at[pl.ds(rbase, ra)],
                                dst.at[b, pl.ds(rbase, ra)])

            @pl.when(last)
            def _():
                pltpu.sync_copy(acc.at[pl.ds((_NS - 1) * ra, rb)],
                                dst.at[b, pl.ds((_NS - 1) * ra, rb)])

    return spmm_kernel(table3, rows2d, cols2d)


def _embed_call(x, w_embed):
    n, feat = x.shape
    hid = w_embed.shape[1]
    nb = hid // _CB
    r = 1000

    def body(x_ref, w_ref, o_ref):
        y = jnp.maximum(
            jnp.dot(x_ref[...], w_ref[...],
                    preferred_element_type=jnp.float32), 0.0)
        for j in range(nb):
            o_ref[j] = y[:, j * _CB:(j + 1) * _CB]

    return pl.pallas_call(
        body,
        grid=(n // r,),
        in_specs=[
            pl.BlockSpec((r, feat), lambda i: (i, 0)),
            pl.BlockSpec((feat, hid), lambda i: (0, 0)),
        ],
        out_specs=pl.BlockSpec((nb, r, _CB), lambda i: (0, i, 0)),
        out_shape=jax.ShapeDtypeStruct((nb, n, _CB), jnp.float32),
    )(x, w_embed)


def _rs1_call(r0, b1, b2):
    nb0, n, cb = r0.shape
    r = 1000

    def body(r0_ref, b1_ref, b2_ref, o_ref):
        j = pl.program_id(1)
        v1 = b1_ref[0] - r0_ref[0]
        v2 = b2_ref[0] - b1_ref[0] - r0_ref[0]
        o_ref[0] = jnp.maximum(jnp.where(j < nb0, v1, v2), 0.0)

    im = lambda i, j: (j % nb0, i, 0)
    return pl.pallas_call(
        body,
        grid=(n // r, 2 * nb0),
        in_specs=[
            pl.BlockSpec((1, r, cb), im),
            pl.BlockSpec((1, r, cb), im),
            pl.BlockSpec((1, r, cb), im),
        ],
        out_specs=pl.BlockSpec((1, r, cb), lambda i, j: (j, i, 0)),
        out_shape=jax.ShapeDtypeStruct((2 * nb0, n, cb), jnp.float32),
    )(r0, b1, b2)


def _final_call(r0, rs1, c1, c2, w_classify):
    n = r0.shape[1]
    cls = w_classify.shape[1]
    r = 1000

    def body(r0_ref, rs1_ref, c1_ref, c2_ref, w_ref, o_ref):
        blocks = [r0_ref[j] for j in range(2)]
        blocks += [rs1_ref[j] for j in range(4)]
        blocks += [jnp.maximum(c1_ref[j] - rs1_ref[j], 0.0) for j in range(4)]
        blocks += [jnp.maximum(c2_ref[j] - c1_ref[j] - rs1_ref[j], 0.0)
                   for j in range(4)]
        acc = jnp.zeros((r, cls), jnp.float32)
        for k, blk in enumerate(blocks):
            acc = acc + jnp.dot(blk, w_ref[k * _CB:(k + 1) * _CB, :],
                                preferred_element_type=jnp.float32)
        m = jnp.max(acc, axis=1, keepdims=True)
        ex = jnp.exp(acc - m)
        o_ref[...] = ex / jnp.sum(ex, axis=1, keepdims=True)

    return pl.pallas_call(
        body,
        grid=(n // r,),
        in_specs=[
            pl.BlockSpec((2, r, _CB), lambda i: (0, i, 0)),
            pl.BlockSpec((4, r, _CB), lambda i: (0, i, 0)),
            pl.BlockSpec((4, r, _CB), lambda i: (0, i, 0)),
            pl.BlockSpec((4, r, _CB), lambda i: (0, i, 0)),
            pl.BlockSpec(w_classify.shape, lambda i: (0, 0)),
        ],
        out_specs=pl.BlockSpec((r, cls), lambda i: (i, 0)),
        out_shape=jax.ShapeDtypeStruct((n, cls), jnp.float32),
    )(r0, rs1, c1, c2, w_classify)


def kernel(x, adj, w_embed, w_classify):
    n = x.shape[0]
    rows2d = adj[0].astype(jnp.int32).reshape(-1, _CHUNK)
    cols2d = adj[1].astype(jnp.int32).reshape(-1, _CHUNK)
    r0 = _embed_call(x, w_embed)
    b1, b2 = _spmm_call(r0, rows2d, cols2d, 2)
    rs1 = _rs1_call(r0, b1, b2)
    c1, c2 = _spmm_call(rs1, rows2d, cols2d, 4)
    return _final_call(r0, rs1, c1, c2, w_classify)
